# Initial kernel scaffold; baseline (speedup 1.0000x reference)
import jax
import jax.numpy as jnp
from jax.experimental import pallas as pl


def kernel(adj, x, w1, b1, w2, b2):
    raise NotImplementedError("write your pallas kernel here")


# speedup vs baseline: 1.8018x; 1.8018x over previous
import functools

import jax
import jax.numpy as jnp
from jax.experimental import pallas as pl
from jax.experimental.pallas import tpu as pltpu


def _round_up(x, m):
    return ((x + m - 1) // m) * m


def _pick_tile(n):
    for t in (512, 256, 128, 64, 32, 16, 8):
        if n % t == 0:
            return t
    return n


def _prep_kernel(adj_ref, x_ref, w1_ref, adjb_ref, d_ref, s1_ref):
    a = adj_ref[...]
    adjb_ref[...] = a.astype(jnp.bfloat16)
    deg = jnp.sum(a, axis=1, keepdims=True) + 1.0
    d_ref[...] = jax.lax.rsqrt(deg)
    s1 = jnp.dot(x_ref[...].astype(jnp.bfloat16), w1_ref[...],
                 preferred_element_type=jnp.float32)
    s1_ref[...] = s1.astype(jnp.bfloat16)


def _layer1_kernel(adjb_ref, s1_ref, d_ref, b1_ref, w2_ref, t2_ref, *, tm):
    i = pl.program_id(0)
    d_all = d_ref[...]
    t1 = (s1_ref[...].astype(jnp.float32) * d_all).astype(jnp.bfloat16)
    acc = jnp.dot(adjb_ref[...], t1, preferred_element_type=jnp.float32)
    start = pl.multiple_of(i * tm, tm)
    d_i = d_ref[pl.ds(start, tm), :]
    t1f_i = s1_ref[pl.ds(start, tm), :].astype(jnp.float32) * d_i
    h = jnp.maximum(d_i * (acc + t1f_i) + b1_ref[...], 0.0)
    s2 = jnp.dot(h.astype(jnp.bfloat16), w2_ref[...],
                 preferred_element_type=jnp.float32)
    t2_ref[...] = (d_i * s2).astype(jnp.bfloat16)


def _layer2_kernel(adjb_ref, t2_ref, d_ref, b2_ref, o_ref, *, tm):
    i = pl.program_id(0)
    acc = jnp.dot(adjb_ref[...], t2_ref[...], preferred_element_type=jnp.float32)
    start = pl.multiple_of(i * tm, tm)
    d_i = d_ref[pl.ds(start, tm), :]
    t2f_i = t2_ref[pl.ds(start, tm), :].astype(jnp.float32)
    o_ref[...] = d_i * (acc + t2f_i) + b2_ref[...]


def kernel(adj, x, w1, b1, w2, b2):
    n = adj.shape[0]
    f_in, h_dim = w1.shape
    c_dim = w2.shape[1]
    fp = _round_up(f_in, 128)
    hp = _round_up(h_dim, 128)
    cp = _round_up(c_dim, 128)
    tm = _pick_tile(n)
    grid = (n // tm,)
    bf16 = jnp.bfloat16
    f32 = jnp.float32

    w1_p = jnp.zeros((fp, hp), bf16).at[:f_in, :h_dim].set(w1.astype(bf16))
    w2_p = jnp.zeros((hp, cp), bf16).at[:h_dim, :c_dim].set(w2.astype(bf16))
    b1_p = jnp.zeros((1, hp), f32).at[0, :h_dim].set(b1)
    b2_p = jnp.zeros((1, cp), f32).at[0, :c_dim].set(b2)
    x_p = x if f_in == fp else jnp.zeros((n, fp), x.dtype).at[:, :f_in].set(x)

    mib = 1 << 20

    adjb, d, s1 = pl.pallas_call(
        _prep_kernel,
        out_shape=(
            jax.ShapeDtypeStruct((n, n), bf16),
            jax.ShapeDtypeStruct((n, 1), f32),
            jax.ShapeDtypeStruct((n, hp), bf16),
        ),
        grid_spec=pltpu.PrefetchScalarGridSpec(
            num_scalar_prefetch=0,
            grid=grid,
            in_specs=[
                pl.BlockSpec((tm, n), lambda i: (i, 0)),
                pl.BlockSpec((tm, fp), lambda i: (i, 0)),
                pl.BlockSpec((fp, hp), lambda i: (0, 0)),
            ],
            out_specs=(
                pl.BlockSpec((tm, n), lambda i: (i, 0)),
                pl.BlockSpec((tm, 1), lambda i: (i, 0)),
                pl.BlockSpec((tm, hp), lambda i: (i, 0)),
            ),
        ),
        compiler_params=pltpu.CompilerParams(
            dimension_semantics=("parallel",),
            vmem_limit_bytes=44 * mib,
        ),
    )(adj, x_p, w1_p)

    t2 = pl.pallas_call(
        functools.partial(_layer1_kernel, tm=tm),
        out_shape=jax.ShapeDtypeStruct((n, cp), bf16),
        grid_spec=pltpu.PrefetchScalarGridSpec(
            num_scalar_prefetch=0,
            grid=grid,
            in_specs=[
                pl.BlockSpec((tm, n), lambda i: (i, 0)),
                pl.BlockSpec((n, hp), lambda i: (0, 0)),
                pl.BlockSpec((n, 1), lambda i: (0, 0)),
                pl.BlockSpec((1, hp), lambda i: (0, 0)),
                pl.BlockSpec((hp, cp), lambda i: (0, 0)),
            ],
            out_specs=pl.BlockSpec((tm, cp), lambda i: (i, 0)),
        ),
        compiler_params=pltpu.CompilerParams(
            dimension_semantics=("parallel",),
            vmem_limit_bytes=32 * mib,
        ),
    )(adjb, s1, d, b1_p, w2_p)

    out_p = pl.pallas_call(
        functools.partial(_layer2_kernel, tm=tm),
        out_shape=jax.ShapeDtypeStruct((n, cp), f32),
        grid_spec=pltpu.PrefetchScalarGridSpec(
            num_scalar_prefetch=0,
            grid=grid,
            in_specs=[
                pl.BlockSpec((tm, n), lambda i: (i, 0)),
                pl.BlockSpec((n, cp), lambda i: (0, 0)),
                pl.BlockSpec((n, 1), lambda i: (0, 0)),
                pl.BlockSpec((1, cp), lambda i: (0, 0)),
            ],
            out_specs=pl.BlockSpec((tm, cp), lambda i: (i, 0)),
        ),
        compiler_params=pltpu.CompilerParams(
            dimension_semantics=("parallel",),
            vmem_limit_bytes=24 * mib,
        ),
    )(adjb, t2, d, b2_p)

    return out_p[:, :c_dim]


# speedup vs baseline: 1.8861x; 1.0468x over previous
import functools

import jax
import jax.numpy as jnp
from jax.experimental import pallas as pl
from jax.experimental.pallas import tpu as pltpu


def _round_up(x, m):
    return ((x + m - 1) // m) * m


def _pick_tile(n):
    for t in (512, 256, 128, 64, 32, 16, 8):
        if n % t == 0:
            return t
    return n


def _prep_kernel(adj_ref, x_ref, w1_ref, adjb_ref, d_ref, s1_ref):
    a = adj_ref[...]
    adjb_ref[...] = a.astype(jnp.int8)
    deg = jnp.sum(a, axis=1, keepdims=True) + 1.0
    d_ref[...] = jax.lax.rsqrt(deg)
    s1 = jnp.dot(x_ref[...].astype(jnp.bfloat16), w1_ref[...],
                 preferred_element_type=jnp.float32)
    s1_ref[...] = s1.astype(jnp.bfloat16)


def _layer1_kernel(adjb_ref, s1_ref, d_ref, b1_ref, w2_ref, t2_ref, *, tm):
    i = pl.program_id(0)
    d_all = d_ref[...]
    t1 = (s1_ref[...].astype(jnp.float32) * d_all).astype(jnp.bfloat16)
    a_blk = adjb_ref[...].astype(jnp.bfloat16)
    acc = jnp.dot(a_blk, t1, preferred_element_type=jnp.float32)
    start = pl.multiple_of(i * tm, tm)
    d_i = d_ref[pl.ds(start, tm), :]
    t1f_i = s1_ref[pl.ds(start, tm), :].astype(jnp.float32) * d_i
    h = jnp.maximum(d_i * (acc + t1f_i) + b1_ref[...], 0.0)
    s2 = jnp.dot(h.astype(jnp.bfloat16), w2_ref[...],
                 preferred_element_type=jnp.float32)
    t2_ref[...] = (d_i * s2).astype(jnp.bfloat16)


def _layer2_kernel(adjb_ref, t2_ref, d_ref, b2_ref, o_ref, *, tm, c_dim):
    i = pl.program_id(0)
    a_blk = adjb_ref[...].astype(jnp.bfloat16)
    acc = jnp.dot(a_blk, t2_ref[...], preferred_element_type=jnp.float32)
    start = pl.multiple_of(i * tm, tm)
    d_i = d_ref[pl.ds(start, tm), :]
    t2f_i = t2_ref[pl.ds(start, tm), :].astype(jnp.float32)
    out = d_i * (acc + t2f_i)
    o_ref[...] = out[:, :c_dim] + b2_ref[...]


def kernel(adj, x, w1, b1, w2, b2):
    n = adj.shape[0]
    f_in, h_dim = w1.shape
    c_dim = w2.shape[1]
    fp = _round_up(f_in, 128)
    hp = _round_up(h_dim, 128)
    cp = _round_up(c_dim, 128)
    tm = _pick_tile(n)
    grid = (n // tm,)
    bf16 = jnp.bfloat16
    f32 = jnp.float32

    w1_p = jnp.zeros((fp, hp), bf16).at[:f_in, :h_dim].set(w1.astype(bf16))
    w2_p = jnp.zeros((hp, cp), bf16).at[:h_dim, :c_dim].set(w2.astype(bf16))
    b1_p = jnp.zeros((1, hp), f32).at[0, :h_dim].set(b1)
    b2_p = b2.reshape(1, c_dim).astype(f32)
    x_p = x if f_in == fp else jnp.zeros((n, fp), x.dtype).at[:, :f_in].set(x)

    mib = 1 << 20

    adjb, d, s1 = pl.pallas_call(
        _prep_kernel,
        out_shape=(
            jax.ShapeDtypeStruct((n, n), jnp.int8),
            jax.ShapeDtypeStruct((n, 1), f32),
            jax.ShapeDtypeStruct((n, hp), bf16),
        ),
        grid_spec=pltpu.PrefetchScalarGridSpec(
            num_scalar_prefetch=0,
            grid=grid,
            in_specs=[
                pl.BlockSpec((tm, n), lambda i: (i, 0)),
                pl.BlockSpec((tm, fp), lambda i: (i, 0)),
                pl.BlockSpec((fp, hp), lambda i: (0, 0)),
            ],
            out_specs=(
                pl.BlockSpec((tm, n), lambda i: (i, 0)),
                pl.BlockSpec((tm, 1), lambda i: (i, 0)),
                pl.BlockSpec((tm, hp), lambda i: (i, 0)),
            ),
        ),
        compiler_params=pltpu.CompilerParams(
            dimension_semantics=("parallel",),
            vmem_limit_bytes=44 * mib,
        ),
    )(adj, x_p, w1_p)

    t2 = pl.pallas_call(
        functools.partial(_layer1_kernel, tm=tm),
        out_shape=jax.ShapeDtypeStruct((n, cp), bf16),
        grid_spec=pltpu.PrefetchScalarGridSpec(
            num_scalar_prefetch=0,
            grid=grid,
            in_specs=[
                pl.BlockSpec((tm, n), lambda i: (i, 0)),
                pl.BlockSpec((n, hp), lambda i: (0, 0)),
                pl.BlockSpec((n, 1), lambda i: (0, 0)),
                pl.BlockSpec((1, hp), lambda i: (0, 0)),
                pl.BlockSpec((hp, cp), lambda i: (0, 0)),
            ],
            out_specs=pl.BlockSpec((tm, cp), lambda i: (i, 0)),
        ),
        compiler_params=pltpu.CompilerParams(
            dimension_semantics=("parallel",),
            vmem_limit_bytes=32 * mib,
        ),
    )(adjb, s1, d, b1_p, w2_p)

    out = pl.pallas_call(
        functools.partial(_layer2_kernel, tm=tm, c_dim=c_dim),
        out_shape=jax.ShapeDtypeStruct((n, c_dim), f32),
        grid_spec=pltpu.PrefetchScalarGridSpec(
            num_scalar_prefetch=0,
            grid=grid,
            in_specs=[
                pl.BlockSpec((tm, n), lambda i: (i, 0)),
                pl.BlockSpec((n, cp), lambda i: (0, 0)),
                pl.BlockSpec((n, 1), lambda i: (0, 0)),
                pl.BlockSpec((1, c_dim), lambda i: (0, 0)),
            ],
            out_specs=pl.BlockSpec((tm, c_dim), lambda i: (i, 0)),
        ),
        compiler_params=pltpu.CompilerParams(
            dimension_semantics=("parallel",),
            vmem_limit_bytes=24 * mib,
        ),
    )(adjb, t2, d, b2_p)

    return out
